# baseline (speedup 1.0000x reference)
import functools

import jax
import jax.numpy as jnp
from jax.experimental import pallas as pl
from jax.experimental.pallas import tpu as pltpu
from jax.experimental.pallas import tpu_sc as plsc

_GATHER_WINDOW = 128
_MM_BLOCK = 512
_DA = 256
_DTAIL = 128


def _sc_gather(emb, tail, idx):
    n = idx.shape[1]

    mesh = plsc.VectorSubcoreMesh(core_axis_name="core", subcore_axis_name="subcore")

    @functools.partial(
        pl.kernel,
        out_type=(
            jax.ShapeDtypeStruct((n, _DA), jnp.float32),
            jax.ShapeDtypeStruct((n, _DTAIL), jnp.float32),
        ),
        mesh=mesh,
    )
    def gather_kernel(x_hbm, t_hbm, i_hbm, oa_hbm, ob_hbm):
        def body(i_vmem, oa_vmem, ob_vmem):
            pltpu.sync_copy(x_hbm.at[i_vmem.at[0], pl.ds(0, _DA)], oa_vmem)
            pltpu.sync_copy(t_hbm.at[i_vmem.at[0]], ob_vmem)

        pltpu.emit_pipeline(
            body,
            grid=(n // _GATHER_WINDOW,),
            in_specs=[pl.BlockSpec((1, _GATHER_WINDOW), lambda i: (0, i))],
            out_specs=[
                pl.BlockSpec((_GATHER_WINDOW, _DA), lambda i: (i, 0)),
                pl.BlockSpec((_GATHER_WINDOW, _DTAIL), lambda i: (i, 0)),
            ],
            core_axis_name=("core", "subcore"),
            dimension_semantics=(pltpu.PARALLEL,),
        )(i_hbm, oa_hbm, ob_hbm)

    return gather_kernel(emb, tail, idx)


def _tc_project(ea, eb, wa, wb, bq2):
    n = ea.shape[0]
    h = wa.shape[1]

    def mm_kernel(ea_ref, eb_ref, wa_ref, wb_ref, b_ref, o_ref):
        o_ref[...] = (
            jnp.dot(ea_ref[...], wa_ref[...], preferred_element_type=jnp.float32)
            + jnp.dot(eb_ref[...], wb_ref[...], preferred_element_type=jnp.float32)
            + b_ref[...]
        )

    return pl.pallas_call(
        mm_kernel,
        grid=(n // _MM_BLOCK,),
        in_specs=[
            pl.BlockSpec((_MM_BLOCK, _DA), lambda i: (i, 0)),
            pl.BlockSpec((_MM_BLOCK, _DTAIL), lambda i: (i, 0)),
            pl.BlockSpec((_DA, h), lambda i: (0, 0)),
            pl.BlockSpec((_DTAIL, h), lambda i: (0, 0)),
            pl.BlockSpec((1, h), lambda i: (0, 0)),
        ],
        out_specs=pl.BlockSpec((_MM_BLOCK, h), lambda i: (i, 0)),
        out_shape=jax.ShapeDtypeStruct((n, h), jnp.float32),
    )(ea, eb, wa, wb, bq2)


def kernel(ques, emb, Wq, bq):
    b, l = ques.shape
    v, d = emb.shape
    h = Wq.shape[1]
    idx = ques.reshape(1, b * l).astype(jnp.int32)
    tail = jnp.zeros((v, _DTAIL), jnp.float32).at[:, : d - _DA].set(emb[:, _DA:])
    wb = jnp.zeros((_DTAIL, h), jnp.float32).at[: d - _DA].set(Wq[_DA:])
    ea, eb = _sc_gather(emb, tail, idx)
    out = _tc_project(ea, eb, Wq[:_DA], wb, bq.reshape(1, h))
    return out.reshape(b, l, h)


# speedup vs baseline: 1.0198x; 1.0198x over previous
import functools

import jax
import jax.numpy as jnp
from jax.experimental import pallas as pl
from jax.experimental.pallas import tpu as pltpu
from jax.experimental.pallas import tpu_sc as plsc

_GATHER_WINDOW = 128
_MM_BLOCK = 512
_LANE = 128
_PAD_BLOCK = 2000


def _tc_pad_tail(emb, d_tail):
    v = emb.shape[0]

    def pad_kernel(x_ref, o_ref):
        mask = jax.lax.broadcasted_iota(jnp.int32, (_PAD_BLOCK, _LANE), 1) < d_tail
        o_ref[...] = jnp.where(mask, x_ref[...], 0.0)

    return pl.pallas_call(
        pad_kernel,
        grid=(v // _PAD_BLOCK,),
        in_specs=[pl.BlockSpec((_PAD_BLOCK, _LANE), lambda i: (i, 2))],
        out_specs=pl.BlockSpec((_PAD_BLOCK, _LANE), lambda i: (i, 0)),
        out_shape=jax.ShapeDtypeStruct((v, _LANE), jnp.float32),
    )(emb)


def _sc_gather(emb, tail, idx):
    n = idx.shape[1]

    mesh = plsc.VectorSubcoreMesh(core_axis_name="core", subcore_axis_name="subcore")

    @functools.partial(
        pl.kernel,
        out_type=(
            jax.ShapeDtypeStruct((n, _LANE), jnp.float32),
            jax.ShapeDtypeStruct((n, _LANE), jnp.float32),
            jax.ShapeDtypeStruct((n, _LANE), jnp.float32),
        ),
        mesh=mesh,
    )
    def gather_kernel(x_hbm, t_hbm, i_hbm, o0_hbm, o1_hbm, o2_hbm):
        def body(i_vmem, o0_vmem, o1_vmem, o2_vmem):
            pltpu.sync_copy(x_hbm.at[i_vmem.at[0], pl.ds(0, _LANE)], o0_vmem)
            pltpu.sync_copy(x_hbm.at[i_vmem.at[0], pl.ds(_LANE, _LANE)], o1_vmem)
            pltpu.sync_copy(t_hbm.at[i_vmem.at[0]], o2_vmem)

        pltpu.emit_pipeline(
            body,
            grid=(n // _GATHER_WINDOW,),
            in_specs=[pl.BlockSpec((1, _GATHER_WINDOW), lambda i: (0, i))],
            out_specs=[
                pl.BlockSpec((_GATHER_WINDOW, _LANE), lambda i: (i, 0)),
                pl.BlockSpec((_GATHER_WINDOW, _LANE), lambda i: (i, 0)),
                pl.BlockSpec((_GATHER_WINDOW, _LANE), lambda i: (i, 0)),
            ],
            core_axis_name=("core", "subcore"),
            dimension_semantics=(pltpu.PARALLEL,),
        )(i_hbm, o0_hbm, o1_hbm, o2_hbm)

    return gather_kernel(emb, tail, idx)


def _tc_project(e0, e1, e2, w0, w1, w2, bq2):
    n = e0.shape[0]
    h = w0.shape[1]

    def mm_kernel(e0_ref, e1_ref, e2_ref, w0_ref, w1_ref, w2_ref, b_ref, o_ref):
        acc = jnp.dot(e0_ref[...], w0_ref[...], preferred_element_type=jnp.float32)
        acc += jnp.dot(e1_ref[...], w1_ref[...], preferred_element_type=jnp.float32)
        acc += jnp.dot(e2_ref[...], w2_ref[...], preferred_element_type=jnp.float32)
        o_ref[...] = acc + b_ref[...]

    return pl.pallas_call(
        mm_kernel,
        grid=(n // _MM_BLOCK,),
        in_specs=[
            pl.BlockSpec((_MM_BLOCK, _LANE), lambda i: (i, 0)),
            pl.BlockSpec((_MM_BLOCK, _LANE), lambda i: (i, 0)),
            pl.BlockSpec((_MM_BLOCK, _LANE), lambda i: (i, 0)),
            pl.BlockSpec((_LANE, h), lambda i: (0, 0)),
            pl.BlockSpec((_LANE, h), lambda i: (0, 0)),
            pl.BlockSpec((_LANE, h), lambda i: (0, 0)),
            pl.BlockSpec((1, h), lambda i: (0, 0)),
        ],
        out_specs=pl.BlockSpec((_MM_BLOCK, h), lambda i: (i, 0)),
        out_shape=jax.ShapeDtypeStruct((n, h), jnp.float32),
    )(e0, e1, e2, w0, w1, w2, bq2)


def kernel(ques, emb, Wq, bq):
    b, l = ques.shape
    v, d = emb.shape
    h = Wq.shape[1]
    d_tail = d - 2 * _LANE
    idx = ques.reshape(1, b * l).astype(jnp.int32)
    tail = _tc_pad_tail(emb, d_tail)
    w2 = jnp.zeros((_LANE, h), jnp.float32).at[:d_tail].set(Wq[2 * _LANE :])
    e0, e1, e2 = _sc_gather(emb, tail, idx)
    out = _tc_project(
        e0, e1, e2, Wq[:_LANE], Wq[_LANE : 2 * _LANE], w2, bq.reshape(1, h)
    )
    return out.reshape(b, l, h)


# speedup vs baseline: 1.5011x; 1.4719x over previous
import functools

import jax
import jax.numpy as jnp
from jax.experimental import pallas as pl
from jax.experimental.pallas import tpu as pltpu
from jax.experimental.pallas import tpu_sc as plsc

_GATHER_WINDOW = 128
_MM_BLOCK = 512
_LANE = 128
_PAD_BLOCK = 2000


def _tc_pad_tail(emb, d_tail):
    v = emb.shape[0]

    def pad_kernel(x_ref, o_ref):
        mask = jax.lax.broadcasted_iota(jnp.int32, (_PAD_BLOCK, _LANE), 1) < d_tail
        o_ref[...] = jnp.where(mask, x_ref[...], 0.0)

    return pl.pallas_call(
        pad_kernel,
        grid=(v // _PAD_BLOCK,),
        in_specs=[pl.BlockSpec((_PAD_BLOCK, _LANE), lambda i: (i, 2))],
        out_specs=pl.BlockSpec((_PAD_BLOCK, _LANE), lambda i: (i, 0)),
        out_shape=jax.ShapeDtypeStruct((v, _LANE), jnp.float32),
    )(emb)


def _sc_gather(emb, tail, idx):
    n = idx.shape[1]

    mesh = plsc.VectorSubcoreMesh(core_axis_name="core", subcore_axis_name="subcore")

    @functools.partial(
        pl.kernel,
        out_type=(
            jax.ShapeDtypeStruct((n, _LANE), jnp.float32),
            jax.ShapeDtypeStruct((n, _LANE), jnp.float32),
            jax.ShapeDtypeStruct((n, _LANE), jnp.float32),
        ),
        mesh=mesh,
    )
    def gather_kernel(x_hbm, t_hbm, i_hbm, o0_hbm, o1_hbm, o2_hbm):
        def body(i_vmem, o0_vmem, o1_vmem, o2_vmem):
            pltpu.sync_copy(x_hbm.at[i_vmem.at[0], pl.ds(0, _LANE)], o0_vmem)
            pltpu.sync_copy(x_hbm.at[i_vmem.at[0], pl.ds(_LANE, _LANE)], o1_vmem)
            pltpu.sync_copy(t_hbm.at[i_vmem.at[0]], o2_vmem)

        pltpu.emit_pipeline(
            body,
            grid=(n // _GATHER_WINDOW,),
            in_specs=[pl.BlockSpec((1, _GATHER_WINDOW), lambda i: (0, i))],
            out_specs=[
                pl.BlockSpec((_GATHER_WINDOW, _LANE), lambda i: (i, 0)),
                pl.BlockSpec((_GATHER_WINDOW, _LANE), lambda i: (i, 0)),
                pl.BlockSpec((_GATHER_WINDOW, _LANE), lambda i: (i, 0)),
            ],
            core_axis_name=("core", "subcore"),
            dimension_semantics=(pltpu.PARALLEL,),
        )(i_hbm, o0_hbm, o1_hbm, o2_hbm)

    return gather_kernel(emb, tail, idx)


def _tc_project(e0, e1, e2, w0, w1, w2, bq2):
    n = e0.shape[0]
    h = w0.shape[1]

    def mm_kernel(e0_ref, e1_ref, e2_ref, w0_ref, w1_ref, w2_ref, b_ref, o_ref):
        acc = jnp.dot(e0_ref[...], w0_ref[...], preferred_element_type=jnp.float32)
        acc += jnp.dot(e1_ref[...], w1_ref[...], preferred_element_type=jnp.float32)
        acc += jnp.dot(e2_ref[...], w2_ref[...], preferred_element_type=jnp.float32)
        o_ref[...] = acc + b_ref[...]

    return pl.pallas_call(
        mm_kernel,
        grid=(n // _MM_BLOCK,),
        in_specs=[
            pl.BlockSpec((_MM_BLOCK, _LANE), lambda i: (i, 0)),
            pl.BlockSpec((_MM_BLOCK, _LANE), lambda i: (i, 0)),
            pl.BlockSpec((_MM_BLOCK, _LANE), lambda i: (i, 0)),
            pl.BlockSpec((_LANE, h), lambda i: (0, 0)),
            pl.BlockSpec((_LANE, h), lambda i: (0, 0)),
            pl.BlockSpec((_LANE, h), lambda i: (0, 0)),
            pl.BlockSpec((1, h), lambda i: (0, 0)),
        ],
        out_specs=pl.BlockSpec((_MM_BLOCK, h), lambda i: (i, 0)),
        out_shape=jax.ShapeDtypeStruct((n, h), jnp.float32),
    )(e0, e1, e2, w0, w1, w2, bq2)


def kernel(ques, emb, Wq, bq):
    b, l = ques.shape
    v, d = emb.shape
    h = Wq.shape[1]
    d_tail = d - 2 * _LANE
    idx = ques.T.reshape(1, b * l).astype(jnp.int32)
    tail = _tc_pad_tail(emb, d_tail)
    w2 = jnp.zeros((_LANE, h), jnp.float32).at[:d_tail].set(Wq[2 * _LANE :])
    e0, e1, e2 = _sc_gather(emb, tail, idx)
    out = _tc_project(
        e0, e1, e2, Wq[:_LANE], Wq[_LANE : 2 * _LANE], w2, bq.reshape(1, h)
    )
    return out.reshape(l, b, h).transpose(1, 0, 2)
